# baseline (speedup 1.0000x reference)
import dataclasses

import jax
import jax.numpy as jnp
from jax import lax
from jax.experimental import pallas as pl
from jax.experimental.pallas import tpu as pltpu
from jax.experimental.pallas import tpu_sc as plsc

N = 10000
D = 128
E = 320000

NC = 2
NS = 16
NW = NC * NS

NP = 10240
HN = NP // NC
B = 128
CW = 160
C = 80
EP = NW * C * B
RS = NP // NS
AR = HN + B
AZ = AR // NS
MR = HN // NS

_mesh = plsc.VectorSubcoreMesh(core_axis_name="c", subcore_axis_name="s")

_cp = pltpu.CompilerParams()
if "needs_layout_passes" in pltpu.CompilerParams.__dataclass_fields__:
    _cp = dataclasses.replace(_cp, needs_layout_passes=False)


def _deg_body(dst_hbm, deg_hbm, idx_v, ones_v, z_v, deg_sh):
    c = lax.axis_index("c")
    s = lax.axis_index("s")
    w = c * NS + s

    @pl.loop(0, B, step=16)
    def _(i):
        ones_v[pl.ds(i, 16)] = jnp.ones((16,), jnp.float32)

    @pl.loop(0, RS, step=16)
    def _(i):
        z_v[pl.ds(i, 16)] = jnp.zeros((16,), jnp.float32)

    pltpu.sync_copy(z_v, deg_sh.at[pl.ds(s * RS, RS)])
    pltpu.sync_copy(dst_hbm.at[w], idx_v)
    plsc.subcore_barrier()

    @pl.loop(0, C)
    def _(j):
        pltpu.sync_copy(ones_v, deg_sh.at[idx_v.at[j]], add=True)

    plsc.subcore_barrier()
    pltpu.sync_copy(deg_sh.at[pl.ds(s * RS, RS)],
                    deg_hbm.at[c, pl.ds(s * RS, RS)])


@jax.jit
def _deg_call(dst3):
    return pl.kernel(
        _deg_body,
        out_type=jax.ShapeDtypeStruct((NC, NP), jnp.float32),
        mesh=_mesh,
        scratch_types=[
            pltpu.VMEM((C, B), jnp.int32),
            pltpu.VMEM((B,), jnp.float32),
            pltpu.VMEM((RS,), jnp.float32),
            pltpu.VMEM_SHARED((NP,), jnp.float32),
        ],
    )(dst3)


K = 4
HC = 80
CWB = CW * B
TRASH = HN
PADSRC = NP - 1


def _route_body(src_hbm, dst_hbm, srcc_hbm, dstc_hbm, ncnt_hbm,
                srcw, dstw, srcb, dstb, nsm):
    c = lax.axis_index("c")
    s = lax.axis_index("s")
    lo = c * HN

    pltpu.sync_copy(src_hbm.at[pl.ds(s * CWB, CWB)], srcw)
    pltpu.sync_copy(dst_hbm.at[pl.ds(s * CWB, CWB)], dstw)

    def body(i, off):
        k = i * 16
        d = dstw[pl.ds(k, 16)]
        sv = srcw[pl.ds(k, 16)]
        m = (d >= lo) & (d < lo + HN) & (d < N)
        plsc.store_compressed(dstb.at[pl.ds(off, 16)], d - lo, mask=m)
        plsc.store_compressed(srcb.at[pl.ds(off, 16)], sv, mask=m)
        return off + jnp.sum(m.astype(jnp.int32))

    off = lax.fori_loop(0, CWB // 16, body, jnp.int32(0))

    for i in range(33):
        dstb[pl.ds(off + 16 * i, 16)] = jnp.full((16,), TRASH, jnp.int32)
        srcb[pl.ds(off + 16 * i, 16)] = jnp.full((16,), PADSRC, jnp.int32)

    nreal = (off + B - 1) // B
    nch = jnp.maximum(((nreal + K - 1) // K) * K, K)
    nsm[pl.ds(0, 16)] = jnp.zeros((16,), jnp.int32) + nch

    pltpu.sync_copy(srcb.at[pl.ds(0, CWB)], srcc_hbm.at[c, s])
    pltpu.sync_copy(dstb.at[pl.ds(0, CWB)], dstc_hbm.at[c, s])
    pltpu.sync_copy(nsm, ncnt_hbm.at[c, s])


@jax.jit
def _route_call(src, dst):
    return pl.kernel(
        _route_body,
        out_type=[jax.ShapeDtypeStruct((NC, NS, CWB), jnp.int32),
                  jax.ShapeDtypeStruct((NC, NS, CWB), jnp.int32),
                  jax.ShapeDtypeStruct((NC, NS, 16), jnp.int32)],
        mesh=_mesh,
        scratch_types=[
            pltpu.VMEM((CWB,), jnp.int32),
            pltpu.VMEM((CWB,), jnp.int32),
            pltpu.VMEM((CWB + 544,), jnp.int32),
            pltpu.VMEM((CWB + 544,), jnp.int32),
            pltpu.VMEM((16,), jnp.int32),
        ],
        compiler_params=_cp,
    )(src, dst)


def _msg_body(y_hbm, srcc_hbm, dstc_hbm, ncnt_hbm, m_hbm,
              srcv, dstv, rows, nsm, acc, gsem, ssem):
    c = lax.axis_index("c")
    s = lax.axis_index("s")

    pltpu.sync_copy(ncnt_hbm.at[c, s], nsm)
    nch = nsm[pl.ds(0, 16)][0]

    @pl.loop(0, B)
    def _(r):
        @pl.loop(0, D, step=16)
        def _(k):
            rows[0, r, pl.ds(k, 16)] = jnp.zeros((16,), jnp.float32)

    pltpu.sync_copy(rows.at[0], acc.at[pl.ds(s * AZ, B)])
    pltpu.sync_copy(rows.at[0], acc.at[pl.ds(s * AZ + B, B)])
    pltpu.sync_copy(rows.at[0, pl.ds(0, AZ - 2 * B)],
                    acc.at[pl.ds(s * AZ + 2 * B, AZ - 2 * B)])
    plsc.subcore_barrier()

    for half in range(2):
        n_here = jnp.clip(nch - half * HC, 0, HC)

        @pl.when(n_here > 0)
        def _():
            pltpu.sync_copy(srcc_hbm.at[c, s, half], srcv)
            pltpu.sync_copy(dstc_hbm.at[c, s, half], dstv)
            for b in range(K):
                pltpu.async_copy(y_hbm.at[srcv.at[b]], rows.at[b],
                                 gsem.at[b])

            def ring(t, _):
                g = t * K
                for b in range(K):
                    j = g + b
                    pltpu.make_async_copy(y_hbm.at[srcv.at[j]], rows.at[b],
                                          gsem.at[b]).wait()
                    pltpu.async_copy(rows.at[b], acc.at[dstv.at[j]],
                                     ssem.at[b], add=True)
                for b in range(K):
                    j = g + b
                    pltpu.make_async_copy(rows.at[b], acc.at[dstv.at[j]],
                                          ssem.at[b]).wait()
                    nxt = jnp.minimum(g + K + b, n_here - 1)
                    pltpu.async_copy(y_hbm.at[srcv.at[nxt]], rows.at[b],
                                     gsem.at[b])
                return 0

            lax.fori_loop(0, n_here // K, ring, 0)

            for b in range(K):
                pltpu.make_async_copy(y_hbm.at[srcv.at[0]], rows.at[b],
                                      gsem.at[b]).wait()

    plsc.subcore_barrier()
    pltpu.sync_copy(acc.at[pl.ds(s * MR, MR)],
                    m_hbm.at[pl.ds(c * HN + s * MR, MR)])


@jax.jit
def _msg_call(y, srcc, dstc, ncnt):
    return pl.kernel(
        _msg_body,
        out_type=jax.ShapeDtypeStruct((NP, D), jnp.float32),
        mesh=_mesh,
        scratch_types=[
            pltpu.VMEM((HC, B), jnp.int32),
            pltpu.VMEM((HC, B), jnp.int32),
            pltpu.VMEM((K, B, D), jnp.float32),
            pltpu.VMEM((16,), jnp.int32),
            pltpu.VMEM_SHARED((AR, D), jnp.float32),
            pltpu.SemaphoreType.DMA((K,)),
            pltpu.SemaphoreType.DMA((K,)),
        ],
    )(y, srcc, dstc, ncnt)


R = 1024
EB = EP // (NP // R)


def _dinv(degp_ref):
    deg = degp_ref[0, :] + degp_ref[1, :] + 1.0
    return lax.rsqrt(jnp.maximum(deg, 1.0))


def _tc_first_body(x_ref, degp_ref, w_ref, y_ref):
    dinv = _dinv(degp_ref)
    xw = lax.dot_general(x_ref[...], w_ref[...], (((1,), (0,)), ((), ())),
                         preferred_element_type=jnp.float32)
    y_ref[...] = xw * dinv[:, None]


def _tc_mid_body(m_ref, yprev_ref, degp_ref, b_ref, w_ref, y_ref):
    dinv = _dinv(degp_ref)
    ssum = m_ref[...] + yprev_ref[...]
    h = jnp.maximum(ssum * dinv[:, None] + b_ref[...], 0.0)
    hw = lax.dot_general(h, w_ref[...], (((1,), (0,)), ((), ())),
                         preferred_element_type=jnp.float32)
    y_ref[...] = hw * dinv[:, None]


def _tc_last_body(m_ref, yprev_ref, degp_ref, b_ref, o_ref):
    dinv = _dinv(degp_ref)
    ssum = m_ref[...] + yprev_ref[...]
    o_ref[...] = ssum * dinv[:, None] + b_ref[...]


_row_spec = pl.BlockSpec((R, D), lambda i: (i, 0))
_deg_spec = pl.BlockSpec((NC, R), lambda i: (0, i))
_b_spec = pl.BlockSpec((1, D), lambda i: (0, 0))
_w_spec = pl.BlockSpec((D, D), lambda i: (0, 0))
_y_sds = jax.ShapeDtypeStruct((NP, D), jnp.float32)


@jax.jit
def _tc_first(xp, degp, W):
    return pl.pallas_call(
        _tc_first_body,
        grid=(NP // R,),
        in_specs=[_row_spec, _deg_spec, _w_spec],
        out_specs=_row_spec,
        out_shape=_y_sds,
    )(xp, degp, W)


@jax.jit
def _tc_mid(m, yprev, degp, b2d, W):
    return pl.pallas_call(
        _tc_mid_body,
        grid=(NP // R,),
        in_specs=[_row_spec, _row_spec, _deg_spec, _b_spec, _w_spec],
        out_specs=_row_spec,
        out_shape=_y_sds,
    )(m, yprev, degp, b2d, W)


@jax.jit
def _tc_last(m, yprev, degp, b2d):
    return pl.pallas_call(
        _tc_last_body,
        grid=(NP // R,),
        in_specs=[_row_spec, _row_spec, _deg_spec, _b_spec],
        out_specs=_row_spec,
        out_shape=_y_sds,
    )(m, yprev, degp, b2d)


def kernel(x, edge_index, W1, b1, W2, b2, W3, b3):
    xp = jnp.zeros((NP, D), jnp.float32).at[:N].set(x)
    pad = jnp.full((EP - E,), NP - 1, jnp.int32)
    src = jnp.concatenate([edge_index[0], pad])
    dst = jnp.concatenate([edge_index[1], pad])
    dst3 = dst.reshape(NW, C, B)

    degp = _deg_call(dst3)
    srcc, dstc, ncnt = _route_call(src, dst)
    srcc = srcc.reshape(NC, NS, 2, HC, B)
    dstc = dstc.reshape(NC, NS, 2, HC, B)

    y1 = _tc_first(xp, degp, W1)
    m1 = _msg_call(y1, srcc, dstc, ncnt)
    y2 = _tc_mid(m1, y1, degp, b1.reshape(1, D), W2)
    m2 = _msg_call(y2, srcc, dstc, ncnt)
    y3 = _tc_mid(m2, y2, degp, b2.reshape(1, D), W3)
    m3 = _msg_call(y3, srcc, dstc, ncnt)
    out = _tc_last(m3, y3, degp, b3.reshape(1, D))
    return out[:N]


# speedup vs baseline: 1.8367x; 1.8367x over previous
import dataclasses

import jax
import jax.numpy as jnp
from jax import lax
from jax.experimental import pallas as pl
from jax.experimental.pallas import tpu as pltpu
from jax.experimental.pallas import tpu_sc as plsc

N = 10000
D = 128
E = 320000

NC = 2
NS = 16
NW = NC * NS

NP = 10240
HN = NP // NC
B = 128
CW = 160
C = 80
EP = NW * C * B
RS = NP // NS
H0 = 4992
AR = HN + B
AZ = AR // NS
MR0 = H0 // NS
MR1 = (NP - H0) // NS

_mesh = plsc.VectorSubcoreMesh(core_axis_name="c", subcore_axis_name="s")

_cp = pltpu.CompilerParams()
if "needs_layout_passes" in pltpu.CompilerParams.__dataclass_fields__:
    _cp = dataclasses.replace(_cp, needs_layout_passes=False)


def _deg_body(dst_hbm, deg_hbm, idx_v, ones_v, z_v, deg_sh):
    c = lax.axis_index("c")
    s = lax.axis_index("s")
    w = c * NS + s

    @pl.loop(0, B, step=16)
    def _(i):
        ones_v[pl.ds(i, 16)] = jnp.ones((16,), jnp.float32)

    @pl.loop(0, RS, step=16)
    def _(i):
        z_v[pl.ds(i, 16)] = jnp.zeros((16,), jnp.float32)

    pltpu.sync_copy(z_v, deg_sh.at[pl.ds(s * RS, RS)])
    pltpu.sync_copy(dst_hbm.at[w], idx_v)
    plsc.subcore_barrier()

    @pl.loop(0, C)
    def _(j):
        pltpu.sync_copy(ones_v, deg_sh.at[idx_v.at[j]], add=True)

    plsc.subcore_barrier()
    pltpu.sync_copy(deg_sh.at[pl.ds(s * RS, RS)],
                    deg_hbm.at[c, pl.ds(s * RS, RS)])


@jax.jit
def _deg_call(dst3):
    return pl.kernel(
        _deg_body,
        out_type=jax.ShapeDtypeStruct((NC, NP), jnp.float32),
        mesh=_mesh,
        scratch_types=[
            pltpu.VMEM((C, B), jnp.int32),
            pltpu.VMEM((B,), jnp.float32),
            pltpu.VMEM((RS,), jnp.float32),
            pltpu.VMEM_SHARED((NP,), jnp.float32),
        ],
    )(dst3)


K = 2
CWB = CW * B
TRASH = HN
PADSRC = NP - 1


def _route_body(src_hbm, dst_hbm, srcc_hbm, dstc_hbm, ncnt_hbm,
                srcw, dstw, srcb, dstb, nsm):
    c = lax.axis_index("c")
    s = lax.axis_index("s")
    lo = c * H0

    pltpu.sync_copy(src_hbm.at[pl.ds(s * CWB, CWB)], srcw)
    pltpu.sync_copy(dst_hbm.at[pl.ds(s * CWB, CWB)], dstw)

    def body(i, off):
        k = i * 16
        d = dstw[pl.ds(k, 16)]
        sv = srcw[pl.ds(k, 16)]
        m = (d >= lo) & (d < lo + H0) & (d < N)
        plsc.store_compressed(dstb.at[pl.ds(off, 16)], d - lo, mask=m)
        plsc.store_compressed(srcb.at[pl.ds(off, 16)], sv, mask=m)
        return off + jnp.sum(m.astype(jnp.int32))

    off = lax.fori_loop(0, CWB // 16, body, jnp.int32(0))

    for i in range(33):
        dstb[pl.ds(off + 16 * i, 16)] = jnp.full((16,), TRASH, jnp.int32)
        srcb[pl.ds(off + 16 * i, 16)] = jnp.full((16,), PADSRC, jnp.int32)

    nreal = (off + B - 1) // B
    nch = jnp.maximum(((nreal + K - 1) // K) * K, K)
    nsm[pl.ds(0, 16)] = jnp.zeros((16,), jnp.int32) + nch

    pltpu.sync_copy(srcb.at[pl.ds(0, CWB)], srcc_hbm.at[c, s])
    pltpu.sync_copy(dstb.at[pl.ds(0, CWB)], dstc_hbm.at[c, s])
    pltpu.sync_copy(nsm, ncnt_hbm.at[c, s])


@jax.jit
def _route_call(src, dst):
    return pl.kernel(
        _route_body,
        out_type=[jax.ShapeDtypeStruct((NC, NS, CWB), jnp.int32),
                  jax.ShapeDtypeStruct((NC, NS, CWB), jnp.int32),
                  jax.ShapeDtypeStruct((NC, NS, 16), jnp.int32)],
        mesh=_mesh,
        scratch_types=[
            pltpu.VMEM((CWB,), jnp.int32),
            pltpu.VMEM((CWB,), jnp.int32),
            pltpu.VMEM((CWB + 544,), jnp.int32),
            pltpu.VMEM((CWB + 544,), jnp.int32),
            pltpu.VMEM((16,), jnp.int32),
        ],
        compiler_params=_cp,
    )(src, dst)


def _msg_body(y_hbm, srcc_hbm, dstc_hbm, ncnt_hbm, m_hbm,
              srcv, dstv, rows, nsm, acc, gsem, ssem):
    c = lax.axis_index("c")
    s = lax.axis_index("s")

    pltpu.sync_copy(ncnt_hbm.at[c, s], nsm)
    nch = nsm[pl.ds(0, 16)][0]
    pltpu.async_copy(srcc_hbm.at[c, s], srcv, gsem.at[0])
    pltpu.async_copy(dstc_hbm.at[c, s], dstv, gsem.at[1])

    @pl.loop(0, B)
    def _(r):
        @pl.loop(0, D, step=16)
        def _(k):
            rows[0, r, pl.ds(k, 16)] = jnp.zeros((16,), jnp.float32)

    pltpu.sync_copy(rows.at[0], acc.at[pl.ds(s * AZ, B)])
    pltpu.sync_copy(rows.at[0], acc.at[pl.ds(s * AZ + B, B)])
    pltpu.sync_copy(rows.at[0, pl.ds(0, AZ - 2 * B)],
                    acc.at[pl.ds(s * AZ + 2 * B, AZ - 2 * B)])
    pltpu.make_async_copy(srcc_hbm.at[c, s], srcv, gsem.at[0]).wait()
    pltpu.make_async_copy(dstc_hbm.at[c, s], dstv, gsem.at[1]).wait()
    plsc.subcore_barrier()

    for b in range(K):
        pltpu.async_copy(y_hbm.at[srcv.at[b]], rows.at[b], gsem.at[b])

    def ring(t, _):
        g = t * K
        for b in range(K):
            j = g + b
            pltpu.make_async_copy(y_hbm.at[srcv.at[j]], rows.at[b],
                                  gsem.at[b]).wait()
            pltpu.async_copy(rows.at[b], acc.at[dstv.at[j]], ssem.at[b],
                             add=True)
        for b in range(K):
            j = g + b
            pltpu.make_async_copy(rows.at[b], acc.at[dstv.at[j]],
                                  ssem.at[b]).wait()
            nxt = jnp.minimum(g + K + b, nch - 1)
            pltpu.async_copy(y_hbm.at[srcv.at[nxt]], rows.at[b], gsem.at[b])
        return 0

    lax.fori_loop(0, nch // K, ring, 0)

    for b in range(K):
        pltpu.make_async_copy(y_hbm.at[srcv.at[0]], rows.at[b],
                              gsem.at[b]).wait()

    plsc.subcore_barrier()

    @pl.when(c == 0)
    def _():
        pltpu.sync_copy(acc.at[pl.ds(s * MR0, MR0)],
                        m_hbm.at[pl.ds(s * MR0, MR0)])

    @pl.when(c == 1)
    def _():
        pltpu.sync_copy(acc.at[pl.ds(s * MR1, MR1)],
                        m_hbm.at[pl.ds(H0 + s * MR1, MR1)])


@jax.jit
def _msg_call(y, srcc, dstc, ncnt):
    return pl.kernel(
        _msg_body,
        out_type=jax.ShapeDtypeStruct((NP, D), jnp.float32),
        mesh=_mesh,
        scratch_types=[
            pltpu.VMEM((CW, B), jnp.int32),
            pltpu.VMEM((CW, B), jnp.int32),
            pltpu.VMEM((K, B, D), jnp.float32),
            pltpu.VMEM((16,), jnp.int32),
            pltpu.VMEM_SHARED((AR, D), jnp.float32),
            pltpu.SemaphoreType.DMA((K,)),
            pltpu.SemaphoreType.DMA((K,)),
        ],
    )(y, srcc, dstc, ncnt)


R = 1024
EB = EP // (NP // R)


def _dinv(degp_ref):
    deg = degp_ref[0, :] + degp_ref[1, :] + 1.0
    return lax.rsqrt(jnp.maximum(deg, 1.0))


def _tc_first_body(x_ref, degp_ref, w_ref, y_ref):
    dinv = _dinv(degp_ref)
    xw = lax.dot_general(x_ref[...], w_ref[...], (((1,), (0,)), ((), ())),
                         preferred_element_type=jnp.float32)
    y_ref[...] = xw * dinv[:, None]


def _tc_mid_body(m_ref, yprev_ref, degp_ref, b_ref, w_ref, y_ref):
    dinv = _dinv(degp_ref)
    ssum = m_ref[...] + yprev_ref[...]
    h = jnp.maximum(ssum * dinv[:, None] + b_ref[...], 0.0)
    hw = lax.dot_general(h, w_ref[...], (((1,), (0,)), ((), ())),
                         preferred_element_type=jnp.float32)
    y_ref[...] = hw * dinv[:, None]


def _tc_last_body(m_ref, yprev_ref, degp_ref, b_ref, o_ref):
    dinv = _dinv(degp_ref)
    ssum = m_ref[...] + yprev_ref[...]
    o_ref[...] = ssum * dinv[:, None] + b_ref[...]


_row_spec = pl.BlockSpec((R, D), lambda i: (i, 0))
_deg_spec = pl.BlockSpec((NC, R), lambda i: (0, i))
_b_spec = pl.BlockSpec((1, D), lambda i: (0, 0))
_w_spec = pl.BlockSpec((D, D), lambda i: (0, 0))
_y_sds = jax.ShapeDtypeStruct((NP, D), jnp.float32)


@jax.jit
def _tc_first(xp, degp, W):
    return pl.pallas_call(
        _tc_first_body,
        grid=(NP // R,),
        in_specs=[_row_spec, _deg_spec, _w_spec],
        out_specs=_row_spec,
        out_shape=_y_sds,
    )(xp, degp, W)


@jax.jit
def _tc_mid(m, yprev, degp, b2d, W):
    return pl.pallas_call(
        _tc_mid_body,
        grid=(NP // R,),
        in_specs=[_row_spec, _row_spec, _deg_spec, _b_spec, _w_spec],
        out_specs=_row_spec,
        out_shape=_y_sds,
    )(m, yprev, degp, b2d, W)


@jax.jit
def _tc_last(m, yprev, degp, b2d):
    return pl.pallas_call(
        _tc_last_body,
        grid=(NP // R,),
        in_specs=[_row_spec, _row_spec, _deg_spec, _b_spec],
        out_specs=_row_spec,
        out_shape=_y_sds,
    )(m, yprev, degp, b2d)


def kernel(x, edge_index, W1, b1, W2, b2, W3, b3):
    xp = jnp.zeros((NP, D), jnp.float32).at[:N].set(x)
    pad = jnp.full((EP - E,), NP - 1, jnp.int32)
    src = jnp.concatenate([edge_index[0], pad])
    dst = jnp.concatenate([edge_index[1], pad])
    dst3 = dst.reshape(NW, C, B)

    degp = _deg_call(dst3)
    srcc, dstc, ncnt = _route_call(src, dst)
    srcc = srcc.reshape(NC, NS, CW, B)
    dstc = dstc.reshape(NC, NS, CW, B)

    y1 = _tc_first(xp, degp, W1)
    m1 = _msg_call(y1, srcc, dstc, ncnt)
    y2 = _tc_mid(m1, y1, degp, b1.reshape(1, D), W2)
    m2 = _msg_call(y2, srcc, dstc, ncnt)
    y3 = _tc_mid(m2, y2, degp, b2.reshape(1, D), W3)
    m3 = _msg_call(y3, srcc, dstc, ncnt)
    out = _tc_last(m3, y3, degp, b3.reshape(1, D))
    return out[:N]


# speedup vs baseline: 1.9796x; 1.0778x over previous
import dataclasses

import jax
import jax.numpy as jnp
from jax import lax
from jax.experimental import pallas as pl
from jax.experimental.pallas import tpu as pltpu
from jax.experimental.pallas import tpu_sc as plsc

N = 10000
D = 128
E = 320000

NC = 2
NS = 16
NW = NC * NS

NP = 10240
HN = NP // NC
B = 128
CW = 160
C = 80
EP = NW * C * B
RS = NP // NS
H0 = 4992
AR = HN + B
AZ = AR // NS
MR0 = H0 // NS
MR1 = (NP - H0) // NS

_mesh = plsc.VectorSubcoreMesh(core_axis_name="c", subcore_axis_name="s")

_cp = pltpu.CompilerParams()
if "needs_layout_passes" in pltpu.CompilerParams.__dataclass_fields__:
    _cp = dataclasses.replace(_cp, needs_layout_passes=False)


def _deg_body(dst_hbm, deg_hbm, idx_v, ones_v, z_v, deg_sh):
    c = lax.axis_index("c")
    s = lax.axis_index("s")
    w = c * NS + s

    @pl.loop(0, B, step=16)
    def _(i):
        ones_v[pl.ds(i, 16)] = jnp.ones((16,), jnp.float32)

    @pl.loop(0, RS, step=16)
    def _(i):
        z_v[pl.ds(i, 16)] = jnp.zeros((16,), jnp.float32)

    pltpu.sync_copy(z_v, deg_sh.at[pl.ds(s * RS, RS)])
    pltpu.sync_copy(dst_hbm.at[w], idx_v)
    plsc.subcore_barrier()

    @pl.loop(0, C)
    def _(j):
        pltpu.sync_copy(ones_v, deg_sh.at[idx_v.at[j]], add=True)

    plsc.subcore_barrier()
    pltpu.sync_copy(deg_sh.at[pl.ds(s * RS, RS)],
                    deg_hbm.at[c, pl.ds(s * RS, RS)])


@jax.jit
def _deg_call(dst3):
    return pl.kernel(
        _deg_body,
        out_type=jax.ShapeDtypeStruct((NC, NP), jnp.float32),
        mesh=_mesh,
        scratch_types=[
            pltpu.VMEM((C, B), jnp.int32),
            pltpu.VMEM((B,), jnp.float32),
            pltpu.VMEM((RS,), jnp.float32),
            pltpu.VMEM_SHARED((NP,), jnp.float32),
        ],
    )(dst3)


K = 2
CWB = CW * B
TRASH = HN
PADSRC = NP - 1


def _route_body(src_hbm, dst_hbm, srcc_hbm, dstc_hbm, ncnt_hbm,
                srcw, dstw, srcb, dstb, nsm):
    c = lax.axis_index("c")
    s = lax.axis_index("s")
    lo = c * H0

    pltpu.sync_copy(src_hbm.at[pl.ds(s * CWB, CWB)], srcw)
    pltpu.sync_copy(dst_hbm.at[pl.ds(s * CWB, CWB)], dstw)

    def body(i, off):
        k = i * 16
        d = dstw[pl.ds(k, 16)]
        sv = srcw[pl.ds(k, 16)]
        hi = jnp.where(c == 0, jnp.int32(H0), jnp.int32(N))
        m = (d >= lo) & (d < hi)
        plsc.store_compressed(dstb.at[pl.ds(off, 16)], d - lo, mask=m)
        plsc.store_compressed(srcb.at[pl.ds(off, 16)], sv, mask=m)
        return off + jnp.sum(m.astype(jnp.int32))

    off = lax.fori_loop(0, CWB // 16, body, jnp.int32(0))

    for i in range(33):
        dstb[pl.ds(off + 16 * i, 16)] = jnp.full((16,), TRASH, jnp.int32)
        srcb[pl.ds(off + 16 * i, 16)] = jnp.full((16,), PADSRC, jnp.int32)

    nreal = (off + B - 1) // B
    nch = jnp.maximum(((nreal + K - 1) // K) * K, K)
    nsm[pl.ds(0, 16)] = jnp.zeros((16,), jnp.int32) + nch

    pltpu.sync_copy(srcb.at[pl.ds(0, CWB)], srcc_hbm.at[c, s])
    pltpu.sync_copy(dstb.at[pl.ds(0, CWB)], dstc_hbm.at[c, s])
    pltpu.sync_copy(nsm, ncnt_hbm.at[c, s])


@jax.jit
def _route_call(src, dst):
    return pl.kernel(
        _route_body,
        out_type=[jax.ShapeDtypeStruct((NC, NS, CWB), jnp.int32),
                  jax.ShapeDtypeStruct((NC, NS, CWB), jnp.int32),
                  jax.ShapeDtypeStruct((NC, NS, 16), jnp.int32)],
        mesh=_mesh,
        scratch_types=[
            pltpu.VMEM((CWB,), jnp.int32),
            pltpu.VMEM((CWB,), jnp.int32),
            pltpu.VMEM((CWB + 544,), jnp.int32),
            pltpu.VMEM((CWB + 544,), jnp.int32),
            pltpu.VMEM((16,), jnp.int32),
        ],
        compiler_params=_cp,
    )(src, dst)


def _msg_body(y_hbm, srcc_hbm, dstc_hbm, ncnt_hbm, m_hbm,
              srcv, dstv, rows, nsm, acc, gsem, ssem):
    c = lax.axis_index("c")
    s = lax.axis_index("s")

    pltpu.sync_copy(ncnt_hbm.at[c, s], nsm)
    nch = nsm[pl.ds(0, 16)][0]
    pltpu.async_copy(srcc_hbm.at[c, s], srcv, gsem.at[0])
    pltpu.async_copy(dstc_hbm.at[c, s], dstv, gsem.at[1])

    @pl.loop(0, B)
    def _(r):
        @pl.loop(0, D, step=16)
        def _(k):
            rows[0, r, pl.ds(k, 16)] = jnp.zeros((16,), jnp.float32)

    pltpu.sync_copy(rows.at[0], acc.at[pl.ds(s * AZ, B)])
    pltpu.sync_copy(rows.at[0], acc.at[pl.ds(s * AZ + B, B)])
    pltpu.sync_copy(rows.at[0, pl.ds(0, AZ - 2 * B)],
                    acc.at[pl.ds(s * AZ + 2 * B, AZ - 2 * B)])
    pltpu.make_async_copy(srcc_hbm.at[c, s], srcv, gsem.at[0]).wait()
    pltpu.make_async_copy(dstc_hbm.at[c, s], dstv, gsem.at[1]).wait()
    plsc.subcore_barrier()

    for b in range(K):
        pltpu.async_copy(y_hbm.at[srcv.at[b]], rows.at[b], gsem.at[b])

    def ring(t, _):
        g = t * K
        for b in range(K):
            j = g + b
            pltpu.make_async_copy(y_hbm.at[srcv.at[j]], rows.at[b],
                                  gsem.at[b]).wait()
            pltpu.async_copy(rows.at[b], acc.at[dstv.at[j]], ssem.at[b],
                             add=True)
        for b in range(K):
            j = g + b
            pltpu.make_async_copy(rows.at[b], acc.at[dstv.at[j]],
                                  ssem.at[b]).wait()
            nxt = jnp.minimum(g + K + b, nch - 1)
            pltpu.async_copy(y_hbm.at[srcv.at[nxt]], rows.at[b], gsem.at[b])
        return 0

    lax.fori_loop(0, nch // K, ring, 0)

    for b in range(K):
        pltpu.make_async_copy(y_hbm.at[srcv.at[0]], rows.at[b],
                              gsem.at[b]).wait()

    plsc.subcore_barrier()

    @pl.when(c == 0)
    def _():
        pltpu.sync_copy(acc.at[pl.ds(s * MR0, MR0)],
                        m_hbm.at[pl.ds(s * MR0, MR0)])

    @pl.when(c == 1)
    def _():
        pltpu.sync_copy(acc.at[pl.ds(s * MR1, MR1)],
                        m_hbm.at[pl.ds(H0 + s * MR1, MR1)])


@jax.jit
def _msg_call(y, srcc, dstc, ncnt):
    return pl.kernel(
        _msg_body,
        out_type=jax.ShapeDtypeStruct((NP, D), jnp.float32),
        mesh=_mesh,
        scratch_types=[
            pltpu.VMEM((CW, B), jnp.int32),
            pltpu.VMEM((CW, B), jnp.int32),
            pltpu.VMEM((K, B, D), jnp.float32),
            pltpu.VMEM((16,), jnp.int32),
            pltpu.VMEM_SHARED((AR, D), jnp.float32),
            pltpu.SemaphoreType.DMA((K,)),
            pltpu.SemaphoreType.DMA((K,)),
        ],
    )(y, srcc, dstc, ncnt)


R = 1024
EB = EP // (NP // R)


def _dinv(degp_ref):
    deg = degp_ref[0, :] + degp_ref[1, :] + 1.0
    return lax.rsqrt(jnp.maximum(deg, 1.0))


def _tc_first_body(x_ref, degp_ref, w_ref, y_ref):
    dinv = _dinv(degp_ref)
    xw = lax.dot_general(x_ref[...], w_ref[...], (((1,), (0,)), ((), ())),
                         preferred_element_type=jnp.float32)
    y_ref[...] = xw * dinv[:, None]


def _tc_mid_body(m_ref, yprev_ref, degp_ref, b_ref, w_ref, y_ref):
    dinv = _dinv(degp_ref)
    ssum = m_ref[...] + yprev_ref[...]
    h = jnp.maximum(ssum * dinv[:, None] + b_ref[...], 0.0)
    hw = lax.dot_general(h, w_ref[...], (((1,), (0,)), ((), ())),
                         preferred_element_type=jnp.float32)
    y_ref[...] = hw * dinv[:, None]


def _tc_last_body(m_ref, yprev_ref, degp_ref, b_ref, o_ref):
    dinv = _dinv(degp_ref)
    ssum = m_ref[...] + yprev_ref[...]
    o_ref[...] = ssum * dinv[:, None] + b_ref[...]


_row_spec = pl.BlockSpec((R, D), lambda i: (i, 0))
_deg_spec = pl.BlockSpec((NC, R), lambda i: (0, i))
_b_spec = pl.BlockSpec((1, D), lambda i: (0, 0))
_w_spec = pl.BlockSpec((D, D), lambda i: (0, 0))
_y_sds = jax.ShapeDtypeStruct((NP, D), jnp.float32)


@jax.jit
def _tc_first(xp, degp, W):
    return pl.pallas_call(
        _tc_first_body,
        grid=(NP // R,),
        in_specs=[_row_spec, _deg_spec, _w_spec],
        out_specs=_row_spec,
        out_shape=_y_sds,
    )(xp, degp, W)


@jax.jit
def _tc_mid(m, yprev, degp, b2d, W):
    return pl.pallas_call(
        _tc_mid_body,
        grid=(NP // R,),
        in_specs=[_row_spec, _row_spec, _deg_spec, _b_spec, _w_spec],
        out_specs=_row_spec,
        out_shape=_y_sds,
    )(m, yprev, degp, b2d, W)


@jax.jit
def _tc_last(m, yprev, degp, b2d):
    return pl.pallas_call(
        _tc_last_body,
        grid=(NP // R,),
        in_specs=[_row_spec, _row_spec, _deg_spec, _b_spec],
        out_specs=_row_spec,
        out_shape=_y_sds,
    )(m, yprev, degp, b2d)


def kernel(x, edge_index, W1, b1, W2, b2, W3, b3):
    xp = jnp.zeros((NP, D), jnp.float32).at[:N].set(x)
    pad = jnp.full((EP - E,), NP - 1, jnp.int32)
    src = jnp.concatenate([edge_index[0], pad])
    dst = jnp.concatenate([edge_index[1], pad])
    dst3 = dst.reshape(NW, C, B)

    degp = _deg_call(dst3)
    srcc, dstc, ncnt = _route_call(src, dst)
    srcc = srcc.reshape(NC, NS, CW, B)
    dstc = dstc.reshape(NC, NS, CW, B)

    y1 = _tc_first(xp, degp, W1)
    m1 = _msg_call(y1, srcc, dstc, ncnt)
    y2 = _tc_mid(m1, y1, degp, b1.reshape(1, D), W2)
    m2 = _msg_call(y2, srcc, dstc, ncnt)
    y3 = _tc_mid(m2, y2, degp, b2.reshape(1, D), W3)
    m3 = _msg_call(y3, srcc, dstc, ncnt)
    out = _tc_last(m3, y3, degp, b3.reshape(1, D))
    return out[:N]


# speedup vs baseline: 2.0690x; 1.0452x over previous
import dataclasses

import jax
import jax.numpy as jnp
from jax import lax
from jax.experimental import pallas as pl
from jax.experimental.pallas import tpu as pltpu
from jax.experimental.pallas import tpu_sc as plsc

N = 10000
D = 128
E = 320000

NC = 2
NS = 16
NW = NC * NS

NP = 10240
HN = NP // NC
B = 128
CW = 160
C = 80
EP = NW * C * B
RS = NP // NS
H0 = 4992
AR = HN + B
AZ = AR // NS
MR0 = H0 // NS
MR1 = (NP - H0) // NS

_mesh = plsc.VectorSubcoreMesh(core_axis_name="c", subcore_axis_name="s")

_cp = pltpu.CompilerParams()
if "needs_layout_passes" in pltpu.CompilerParams.__dataclass_fields__:
    _cp = dataclasses.replace(_cp, needs_layout_passes=False)


def _deg_body(dst_hbm, deg_hbm, idx_v, ones_v, z_v, deg_sh):
    c = lax.axis_index("c")
    s = lax.axis_index("s")
    w = c * NS + s

    @pl.loop(0, B, step=16)
    def _(i):
        ones_v[pl.ds(i, 16)] = jnp.ones((16,), jnp.float32)

    @pl.loop(0, RS, step=16)
    def _(i):
        z_v[pl.ds(i, 16)] = jnp.zeros((16,), jnp.float32)

    pltpu.sync_copy(z_v, deg_sh.at[pl.ds(s * RS, RS)])
    pltpu.sync_copy(dst_hbm.at[w], idx_v)
    plsc.subcore_barrier()

    @pl.loop(0, C)
    def _(j):
        pltpu.sync_copy(ones_v, deg_sh.at[idx_v.at[j]], add=True)

    plsc.subcore_barrier()
    pltpu.sync_copy(deg_sh.at[pl.ds(s * RS, RS)],
                    deg_hbm.at[c, pl.ds(s * RS, RS)])


@jax.jit
def _deg_call(dst3):
    return pl.kernel(
        _deg_body,
        out_type=jax.ShapeDtypeStruct((NC, NP), jnp.float32),
        mesh=_mesh,
        scratch_types=[
            pltpu.VMEM((C, B), jnp.int32),
            pltpu.VMEM((B,), jnp.float32),
            pltpu.VMEM((RS,), jnp.float32),
            pltpu.VMEM_SHARED((NP,), jnp.float32),
        ],
    )(dst3)


K = 2
CWB = CW * B
TRASH = HN
PADSRC = NP - 1


def _route_body(src_hbm, dst_hbm, srcc_hbm, dstc_hbm, ncnt_hbm,
                srcw, dstw, srcb, dstb, nsm):
    c = lax.axis_index("c")
    s = lax.axis_index("s")
    lo = c * H0

    pltpu.sync_copy(src_hbm.at[pl.ds(s * CWB, CWB)], srcw)
    pltpu.sync_copy(dst_hbm.at[pl.ds(s * CWB, CWB)], dstw)

    def body(i, off):
        k = i * 16
        d = dstw[pl.ds(k, 16)]
        sv = srcw[pl.ds(k, 16)]
        hi = jnp.where(c == 0, jnp.int32(H0), jnp.int32(N))
        m = (d >= lo) & (d < hi)
        plsc.store_compressed(dstb.at[pl.ds(off, 16)], d - lo, mask=m)
        plsc.store_compressed(srcb.at[pl.ds(off, 16)], sv, mask=m)
        return off + jnp.sum(m.astype(jnp.int32))

    off = lax.fori_loop(0, CWB // 16, body, jnp.int32(0))

    for i in range(33):
        dstb[pl.ds(off + 16 * i, 16)] = jnp.full((16,), TRASH, jnp.int32)
        srcb[pl.ds(off + 16 * i, 16)] = jnp.full((16,), PADSRC, jnp.int32)

    nreal = (off + B - 1) // B
    nch = jnp.maximum(((nreal + K - 1) // K) * K, K)
    nsm[pl.ds(0, 16)] = jnp.zeros((16,), jnp.int32) + nch

    pltpu.sync_copy(srcb.at[pl.ds(0, CWB)], srcc_hbm.at[c, s])
    pltpu.sync_copy(dstb.at[pl.ds(0, CWB)], dstc_hbm.at[c, s])
    pltpu.sync_copy(nsm, ncnt_hbm.at[c, s])


@jax.jit
def _route_call(src, dst):
    return pl.kernel(
        _route_body,
        out_type=[jax.ShapeDtypeStruct((NC, NS, CWB), jnp.int32),
                  jax.ShapeDtypeStruct((NC, NS, CWB), jnp.int32),
                  jax.ShapeDtypeStruct((NC, NS, 16), jnp.int32)],
        mesh=_mesh,
        scratch_types=[
            pltpu.VMEM((CWB,), jnp.int32),
            pltpu.VMEM((CWB,), jnp.int32),
            pltpu.VMEM((CWB + 544,), jnp.int32),
            pltpu.VMEM((CWB + 544,), jnp.int32),
            pltpu.VMEM((16,), jnp.int32),
        ],
        compiler_params=_cp,
    )(src, dst)


def _msg_body(y_hbm, srcc_hbm, dstc_hbm, ncnt_hbm, m_hbm,
              srcv, dstv, rows, nsm, acc, gsem, ssem):
    c = lax.axis_index("c")
    s = lax.axis_index("s")

    pltpu.sync_copy(ncnt_hbm.at[c, s], nsm)
    nch = nsm[pl.ds(0, 16)][0]
    pltpu.async_copy(srcc_hbm.at[c, s], srcv, gsem.at[0])
    pltpu.async_copy(dstc_hbm.at[c, s], dstv, gsem.at[1])

    @pl.loop(0, B)
    def _(r):
        @pl.loop(0, D, step=16)
        def _(k):
            rows[0, r, pl.ds(k, 16)] = jnp.zeros((16,), jnp.float32)

    pltpu.sync_copy(rows.at[0], acc.at[pl.ds(s * AZ, B)])
    pltpu.sync_copy(rows.at[0], acc.at[pl.ds(s * AZ + B, B)])
    pltpu.sync_copy(rows.at[0, pl.ds(0, AZ - 2 * B)],
                    acc.at[pl.ds(s * AZ + 2 * B, AZ - 2 * B)])
    pltpu.make_async_copy(srcc_hbm.at[c, s], srcv, gsem.at[0]).wait()
    pltpu.make_async_copy(dstc_hbm.at[c, s], dstv, gsem.at[1]).wait()
    plsc.subcore_barrier()

    for b in range(K):
        pltpu.async_copy(y_hbm.at[srcv.at[b]], rows.at[b], gsem.at[b])

    def ring(t, _):
        g = t * K
        for b in range(K):
            j = g + b
            pltpu.make_async_copy(y_hbm.at[srcv.at[j]], rows.at[b],
                                  gsem.at[b]).wait()
            pltpu.sync_copy(rows.at[b], acc.at[dstv.at[j]], add=True)
            nxt = jnp.minimum(g + K + b, nch - 1)
            pltpu.async_copy(y_hbm.at[srcv.at[nxt]], rows.at[b], gsem.at[b])
        return 0

    lax.fori_loop(0, nch // K, ring, 0)

    for b in range(K):
        pltpu.make_async_copy(y_hbm.at[srcv.at[0]], rows.at[b],
                              gsem.at[b]).wait()

    plsc.subcore_barrier()

    @pl.when(c == 0)
    def _():
        pltpu.sync_copy(acc.at[pl.ds(s * MR0, MR0)],
                        m_hbm.at[pl.ds(s * MR0, MR0)])

    @pl.when(c == 1)
    def _():
        pltpu.sync_copy(acc.at[pl.ds(s * MR1, MR1)],
                        m_hbm.at[pl.ds(H0 + s * MR1, MR1)])


@jax.jit
def _msg_call(y, srcc, dstc, ncnt):
    return pl.kernel(
        _msg_body,
        out_type=jax.ShapeDtypeStruct((NP, D), jnp.float32),
        mesh=_mesh,
        scratch_types=[
            pltpu.VMEM((CW, B), jnp.int32),
            pltpu.VMEM((CW, B), jnp.int32),
            pltpu.VMEM((K, B, D), jnp.float32),
            pltpu.VMEM((16,), jnp.int32),
            pltpu.VMEM_SHARED((AR, D), jnp.float32),
            pltpu.SemaphoreType.DMA((K,)),
            pltpu.SemaphoreType.DMA((K,)),
        ],
    )(y, srcc, dstc, ncnt)


R = 1024
EB = EP // (NP // R)


def _dinv(degp_ref):
    deg = degp_ref[0, :] + degp_ref[1, :] + 1.0
    return lax.rsqrt(jnp.maximum(deg, 1.0))


def _tc_first_body(x_ref, degp_ref, w_ref, y_ref):
    dinv = _dinv(degp_ref)
    xw = lax.dot_general(x_ref[...], w_ref[...], (((1,), (0,)), ((), ())),
                         preferred_element_type=jnp.float32)
    y_ref[...] = xw * dinv[:, None]


def _tc_mid_body(m_ref, yprev_ref, degp_ref, b_ref, w_ref, y_ref):
    dinv = _dinv(degp_ref)
    ssum = m_ref[...] + yprev_ref[...]
    h = jnp.maximum(ssum * dinv[:, None] + b_ref[...], 0.0)
    hw = lax.dot_general(h, w_ref[...], (((1,), (0,)), ((), ())),
                         preferred_element_type=jnp.float32)
    y_ref[...] = hw * dinv[:, None]


def _tc_last_body(m_ref, yprev_ref, degp_ref, b_ref, o_ref):
    dinv = _dinv(degp_ref)
    ssum = m_ref[...] + yprev_ref[...]
    o_ref[...] = ssum * dinv[:, None] + b_ref[...]


_row_spec = pl.BlockSpec((R, D), lambda i: (i, 0))
_deg_spec = pl.BlockSpec((NC, R), lambda i: (0, i))
_b_spec = pl.BlockSpec((1, D), lambda i: (0, 0))
_w_spec = pl.BlockSpec((D, D), lambda i: (0, 0))
_y_sds = jax.ShapeDtypeStruct((NP, D), jnp.float32)


@jax.jit
def _tc_first(xp, degp, W):
    return pl.pallas_call(
        _tc_first_body,
        grid=(NP // R,),
        in_specs=[_row_spec, _deg_spec, _w_spec],
        out_specs=_row_spec,
        out_shape=_y_sds,
    )(xp, degp, W)


@jax.jit
def _tc_mid(m, yprev, degp, b2d, W):
    return pl.pallas_call(
        _tc_mid_body,
        grid=(NP // R,),
        in_specs=[_row_spec, _row_spec, _deg_spec, _b_spec, _w_spec],
        out_specs=_row_spec,
        out_shape=_y_sds,
    )(m, yprev, degp, b2d, W)


@jax.jit
def _tc_last(m, yprev, degp, b2d):
    return pl.pallas_call(
        _tc_last_body,
        grid=(NP // R,),
        in_specs=[_row_spec, _row_spec, _deg_spec, _b_spec],
        out_specs=_row_spec,
        out_shape=_y_sds,
    )(m, yprev, degp, b2d)


def kernel(x, edge_index, W1, b1, W2, b2, W3, b3):
    xp = jnp.zeros((NP, D), jnp.float32).at[:N].set(x)
    pad = jnp.full((EP - E,), NP - 1, jnp.int32)
    src = jnp.concatenate([edge_index[0], pad])
    dst = jnp.concatenate([edge_index[1], pad])
    dst3 = dst.reshape(NW, C, B)

    degp = _deg_call(dst3)
    srcc, dstc, ncnt = _route_call(src, dst)
    srcc = srcc.reshape(NC, NS, CW, B)
    dstc = dstc.reshape(NC, NS, CW, B)

    y1 = _tc_first(xp, degp, W1)
    m1 = _msg_call(y1, srcc, dstc, ncnt)
    y2 = _tc_mid(m1, y1, degp, b1.reshape(1, D), W2)
    m2 = _msg_call(y2, srcc, dstc, ncnt)
    y3 = _tc_mid(m2, y2, degp, b2.reshape(1, D), W3)
    m3 = _msg_call(y3, srcc, dstc, ncnt)
    out = _tc_last(m3, y3, degp, b3.reshape(1, D))
    return out[:N]


# speedup vs baseline: 2.1056x; 1.0177x over previous
import dataclasses

import jax
import jax.numpy as jnp
from jax import lax
from jax.experimental import pallas as pl
from jax.experimental.pallas import tpu as pltpu
from jax.experimental.pallas import tpu_sc as plsc

N = 10000
D = 128
E = 320000

NC = 2
NS = 16
NW = NC * NS

NP = 10240
HN = NP // NC
B = 128
CW = 160
C = 80
EP = NW * C * B
RS = NP // NS
H0 = 4608
AR = 5632
AZ = AR // NS
MR0 = H0 // NS
MR1 = (NP - H0) // NS

_mesh = plsc.VectorSubcoreMesh(core_axis_name="c", subcore_axis_name="s")

_cp = pltpu.CompilerParams()
if "needs_layout_passes" in pltpu.CompilerParams.__dataclass_fields__:
    _cp = dataclasses.replace(_cp, needs_layout_passes=False)


def _deg_body(dst_hbm, deg_hbm, idx_v, ones_v, z_v, deg_sh):
    c = lax.axis_index("c")
    s = lax.axis_index("s")
    w = c * NS + s

    @pl.loop(0, B, step=16)
    def _(i):
        ones_v[pl.ds(i, 16)] = jnp.ones((16,), jnp.float32)

    @pl.loop(0, RS, step=16)
    def _(i):
        z_v[pl.ds(i, 16)] = jnp.zeros((16,), jnp.float32)

    pltpu.sync_copy(z_v, deg_sh.at[pl.ds(s * RS, RS)])
    pltpu.sync_copy(dst_hbm.at[w], idx_v)
    plsc.subcore_barrier()

    @pl.loop(0, C)
    def _(j):
        pltpu.sync_copy(ones_v, deg_sh.at[idx_v.at[j]], add=True)

    plsc.subcore_barrier()
    pltpu.sync_copy(deg_sh.at[pl.ds(s * RS, RS)],
                    deg_hbm.at[c, pl.ds(s * RS, RS)])


@jax.jit
def _deg_call(dst3):
    return pl.kernel(
        _deg_body,
        out_type=jax.ShapeDtypeStruct((NC, NP), jnp.float32),
        mesh=_mesh,
        scratch_types=[
            pltpu.VMEM((C, B), jnp.int32),
            pltpu.VMEM((B,), jnp.float32),
            pltpu.VMEM((RS,), jnp.float32),
            pltpu.VMEM_SHARED((NP,), jnp.float32),
        ],
    )(dst3)


K = 2
CWB = CW * B
TRASH = 5504
PADSRC = NP - 1


def _route_body(src_hbm, dst_hbm, srcc_hbm, dstc_hbm, ncnt_hbm,
                srcw, dstw, srcb, dstb, nsm):
    c = lax.axis_index("c")
    s = lax.axis_index("s")
    lo = c * H0

    pltpu.sync_copy(src_hbm.at[pl.ds(s * CWB, CWB)], srcw)
    pltpu.sync_copy(dst_hbm.at[pl.ds(s * CWB, CWB)], dstw)

    def body(i, off):
        k = i * 16
        d = dstw[pl.ds(k, 16)]
        sv = srcw[pl.ds(k, 16)]
        hi = jnp.where(c == 0, jnp.int32(H0), jnp.int32(N))
        m = (d >= lo) & (d < hi)
        plsc.store_compressed(dstb.at[pl.ds(off, 16)], d - lo, mask=m)
        plsc.store_compressed(srcb.at[pl.ds(off, 16)], sv, mask=m)
        return off + jnp.sum(m.astype(jnp.int32))

    off = lax.fori_loop(0, CWB // 16, body, jnp.int32(0))

    for i in range(33):
        dstb[pl.ds(off + 16 * i, 16)] = jnp.full((16,), TRASH, jnp.int32)
        srcb[pl.ds(off + 16 * i, 16)] = jnp.full((16,), PADSRC, jnp.int32)

    nreal = (off + B - 1) // B
    nch = jnp.maximum(((nreal + K - 1) // K) * K, K)
    nsm[pl.ds(0, 16)] = jnp.zeros((16,), jnp.int32) + nch

    pltpu.sync_copy(srcb.at[pl.ds(0, CWB)], srcc_hbm.at[c, s])
    pltpu.sync_copy(dstb.at[pl.ds(0, CWB)], dstc_hbm.at[c, s])
    pltpu.sync_copy(nsm, ncnt_hbm.at[c, s])


@jax.jit
def _route_call(src, dst):
    return pl.kernel(
        _route_body,
        out_type=[jax.ShapeDtypeStruct((NC, NS, CWB), jnp.int32),
                  jax.ShapeDtypeStruct((NC, NS, CWB), jnp.int32),
                  jax.ShapeDtypeStruct((NC, NS, 16), jnp.int32)],
        mesh=_mesh,
        scratch_types=[
            pltpu.VMEM((CWB,), jnp.int32),
            pltpu.VMEM((CWB,), jnp.int32),
            pltpu.VMEM((CWB + 544,), jnp.int32),
            pltpu.VMEM((CWB + 544,), jnp.int32),
            pltpu.VMEM((16,), jnp.int32),
        ],
        compiler_params=_cp,
    )(src, dst)


def _msg_body(y_hbm, srcc_hbm, dstc_hbm, ncnt_hbm, m_hbm,
              srcv, dstv, rows, nsm, acc, gsem, ssem):
    c = lax.axis_index("c")
    s = lax.axis_index("s")

    pltpu.sync_copy(ncnt_hbm.at[c, s], nsm)
    nch = nsm[pl.ds(0, 16)][0]
    pltpu.async_copy(srcc_hbm.at[c, s], srcv, gsem.at[0])
    pltpu.async_copy(dstc_hbm.at[c, s], dstv, gsem.at[1])

    @pl.loop(0, B)
    def _(r):
        @pl.loop(0, D, step=16)
        def _(k):
            rows[0, r, pl.ds(k, 16)] = jnp.zeros((16,), jnp.float32)

    pltpu.sync_copy(rows.at[0], acc.at[pl.ds(s * AZ, B)])
    pltpu.sync_copy(rows.at[0], acc.at[pl.ds(s * AZ + B, B)])
    pltpu.sync_copy(rows.at[0, pl.ds(0, AZ - 2 * B)],
                    acc.at[pl.ds(s * AZ + 2 * B, AZ - 2 * B)])
    pltpu.make_async_copy(srcc_hbm.at[c, s], srcv, gsem.at[0]).wait()
    pltpu.make_async_copy(dstc_hbm.at[c, s], dstv, gsem.at[1]).wait()
    plsc.subcore_barrier()

    for b in range(K):
        pltpu.async_copy(y_hbm.at[srcv.at[b]], rows.at[b], gsem.at[b])

    def ring(t, _):
        g = t * K
        for b in range(K):
            j = g + b
            pltpu.make_async_copy(y_hbm.at[srcv.at[j]], rows.at[b],
                                  gsem.at[b]).wait()
            pltpu.sync_copy(rows.at[b], acc.at[dstv.at[j]], add=True)
            nxt = jnp.minimum(g + K + b, nch - 1)
            pltpu.async_copy(y_hbm.at[srcv.at[nxt]], rows.at[b], gsem.at[b])
        return 0

    lax.fori_loop(0, nch // K, ring, 0)

    for b in range(K):
        pltpu.make_async_copy(y_hbm.at[srcv.at[0]], rows.at[b],
                              gsem.at[b]).wait()

    plsc.subcore_barrier()

    @pl.when(c == 0)
    def _():
        pltpu.sync_copy(acc.at[pl.ds(s * MR0, MR0)],
                        m_hbm.at[pl.ds(s * MR0, MR0)])

    @pl.when(c == 1)
    def _():
        pltpu.sync_copy(acc.at[pl.ds(s * MR1, MR1)],
                        m_hbm.at[pl.ds(H0 + s * MR1, MR1)])


@jax.jit
def _msg_call(y, srcc, dstc, ncnt):
    return pl.kernel(
        _msg_body,
        out_type=jax.ShapeDtypeStruct((NP, D), jnp.float32),
        mesh=_mesh,
        scratch_types=[
            pltpu.VMEM((CW, B), jnp.int32),
            pltpu.VMEM((CW, B), jnp.int32),
            pltpu.VMEM((K, B, D), jnp.float32),
            pltpu.VMEM((16,), jnp.int32),
            pltpu.VMEM_SHARED((AR, D), jnp.float32),
            pltpu.SemaphoreType.DMA((K,)),
            pltpu.SemaphoreType.DMA((K,)),
        ],
    )(y, srcc, dstc, ncnt)


R = 1024
EB = EP // (NP // R)


def _dinv(degp_ref):
    deg = degp_ref[0, :] + degp_ref[1, :] + 1.0
    return lax.rsqrt(jnp.maximum(deg, 1.0))


def _tc_first_body(x_ref, degp_ref, w_ref, y_ref):
    dinv = _dinv(degp_ref)
    xw = lax.dot_general(x_ref[...], w_ref[...], (((1,), (0,)), ((), ())),
                         preferred_element_type=jnp.float32)
    y_ref[...] = xw * dinv[:, None]


def _tc_mid_body(m_ref, yprev_ref, degp_ref, b_ref, w_ref, y_ref):
    dinv = _dinv(degp_ref)
    ssum = m_ref[...] + yprev_ref[...]
    h = jnp.maximum(ssum * dinv[:, None] + b_ref[...], 0.0)
    hw = lax.dot_general(h, w_ref[...], (((1,), (0,)), ((), ())),
                         preferred_element_type=jnp.float32)
    y_ref[...] = hw * dinv[:, None]


def _tc_last_body(m_ref, yprev_ref, degp_ref, b_ref, o_ref):
    dinv = _dinv(degp_ref)
    ssum = m_ref[...] + yprev_ref[...]
    o_ref[...] = ssum * dinv[:, None] + b_ref[...]


_row_spec = pl.BlockSpec((R, D), lambda i: (i, 0))
_deg_spec = pl.BlockSpec((NC, R), lambda i: (0, i))
_b_spec = pl.BlockSpec((1, D), lambda i: (0, 0))
_w_spec = pl.BlockSpec((D, D), lambda i: (0, 0))
_y_sds = jax.ShapeDtypeStruct((NP, D), jnp.float32)


@jax.jit
def _tc_first(xp, degp, W):
    return pl.pallas_call(
        _tc_first_body,
        grid=(NP // R,),
        in_specs=[_row_spec, _deg_spec, _w_spec],
        out_specs=_row_spec,
        out_shape=_y_sds,
    )(xp, degp, W)


@jax.jit
def _tc_mid(m, yprev, degp, b2d, W):
    return pl.pallas_call(
        _tc_mid_body,
        grid=(NP // R,),
        in_specs=[_row_spec, _row_spec, _deg_spec, _b_spec, _w_spec],
        out_specs=_row_spec,
        out_shape=_y_sds,
    )(m, yprev, degp, b2d, W)


@jax.jit
def _tc_last(m, yprev, degp, b2d):
    return pl.pallas_call(
        _tc_last_body,
        grid=(NP // R,),
        in_specs=[_row_spec, _row_spec, _deg_spec, _b_spec],
        out_specs=_row_spec,
        out_shape=_y_sds,
    )(m, yprev, degp, b2d)


def kernel(x, edge_index, W1, b1, W2, b2, W3, b3):
    xp = jnp.zeros((NP, D), jnp.float32).at[:N].set(x)
    pad = jnp.full((EP - E,), NP - 1, jnp.int32)
    src = jnp.concatenate([edge_index[0], pad])
    dst = jnp.concatenate([edge_index[1], pad])
    dst3 = dst.reshape(NW, C, B)

    degp = _deg_call(dst3)
    srcc, dstc, ncnt = _route_call(src, dst)
    srcc = srcc.reshape(NC, NS, CW, B)
    dstc = dstc.reshape(NC, NS, CW, B)

    y1 = _tc_first(xp, degp, W1)
    m1 = _msg_call(y1, srcc, dstc, ncnt)
    y2 = _tc_mid(m1, y1, degp, b1.reshape(1, D), W2)
    m2 = _msg_call(y2, srcc, dstc, ncnt)
    y3 = _tc_mid(m2, y2, degp, b2.reshape(1, D), W3)
    m3 = _msg_call(y3, srcc, dstc, ncnt)
    out = _tc_last(m3, y3, degp, b3.reshape(1, D))
    return out[:N]


# speedup vs baseline: 2.1335x; 1.0133x over previous
import dataclasses

import jax
import jax.numpy as jnp
from jax import lax
from jax.experimental import pallas as pl
from jax.experimental.pallas import tpu as pltpu
from jax.experimental.pallas import tpu_sc as plsc

N = 10000
D = 128
E = 320000

NC = 2
NS = 16
NW = NC * NS

NP = 10240
HN = NP // NC
B = 128
CW = 160
C = 80
EP = NW * C * B
RS = NP // NS
H0 = 4480
AR = 5760
AZ = AR // NS
MR0 = H0 // NS
MR1 = (NP - H0) // NS

_mesh = plsc.VectorSubcoreMesh(core_axis_name="c", subcore_axis_name="s")

_cp = pltpu.CompilerParams()
if "needs_layout_passes" in pltpu.CompilerParams.__dataclass_fields__:
    _cp = dataclasses.replace(_cp, needs_layout_passes=False)


def _deg_body(dst_hbm, deg_hbm, idx_v, ones_v, z_v, deg_sh):
    c = lax.axis_index("c")
    s = lax.axis_index("s")
    w = c * NS + s

    @pl.loop(0, B, step=16)
    def _(i):
        ones_v[pl.ds(i, 16)] = jnp.ones((16,), jnp.float32)

    @pl.loop(0, RS, step=16)
    def _(i):
        z_v[pl.ds(i, 16)] = jnp.zeros((16,), jnp.float32)

    pltpu.sync_copy(z_v, deg_sh.at[pl.ds(s * RS, RS)])
    pltpu.sync_copy(dst_hbm.at[w], idx_v)
    plsc.subcore_barrier()

    @pl.loop(0, C)
    def _(j):
        pltpu.sync_copy(ones_v, deg_sh.at[idx_v.at[j]], add=True)

    plsc.subcore_barrier()
    pltpu.sync_copy(deg_sh.at[pl.ds(s * RS, RS)],
                    deg_hbm.at[c, pl.ds(s * RS, RS)])


@jax.jit
def _deg_call(dst3):
    return pl.kernel(
        _deg_body,
        out_type=jax.ShapeDtypeStruct((NC, NP), jnp.float32),
        mesh=_mesh,
        scratch_types=[
            pltpu.VMEM((C, B), jnp.int32),
            pltpu.VMEM((B,), jnp.float32),
            pltpu.VMEM((RS,), jnp.float32),
            pltpu.VMEM_SHARED((NP,), jnp.float32),
        ],
    )(dst3)


K = 2
CWB = CW * B
TRASH = 5632
PADSRC = NP - 1


def _route_body(src_hbm, dst_hbm, srcc_hbm, dstc_hbm, ncnt_hbm,
                srcw, dstw, srcb, dstb, nsm):
    c = lax.axis_index("c")
    s = lax.axis_index("s")
    lo = c * H0

    pltpu.sync_copy(src_hbm.at[pl.ds(s * CWB, CWB)], srcw)
    pltpu.sync_copy(dst_hbm.at[pl.ds(s * CWB, CWB)], dstw)

    def body(i, off):
        k = i * 16
        d = dstw[pl.ds(k, 16)]
        sv = srcw[pl.ds(k, 16)]
        hi = jnp.where(c == 0, jnp.int32(H0), jnp.int32(N))
        m = (d >= lo) & (d < hi)
        plsc.store_compressed(dstb.at[pl.ds(off, 16)], d - lo, mask=m)
        plsc.store_compressed(srcb.at[pl.ds(off, 16)], sv, mask=m)
        return off + jnp.sum(m.astype(jnp.int32))

    off = lax.fori_loop(0, CWB // 16, body, jnp.int32(0))

    for i in range(33):
        dstb[pl.ds(off + 16 * i, 16)] = jnp.full((16,), TRASH, jnp.int32)
        srcb[pl.ds(off + 16 * i, 16)] = jnp.full((16,), PADSRC, jnp.int32)

    nreal = (off + B - 1) // B
    nch = jnp.maximum(((nreal + K - 1) // K) * K, K)
    nsm[pl.ds(0, 16)] = jnp.zeros((16,), jnp.int32) + nch

    pltpu.sync_copy(srcb.at[pl.ds(0, CWB)], srcc_hbm.at[c, s])
    pltpu.sync_copy(dstb.at[pl.ds(0, CWB)], dstc_hbm.at[c, s])
    pltpu.sync_copy(nsm, ncnt_hbm.at[c, s])


@jax.jit
def _route_call(src, dst):
    return pl.kernel(
        _route_body,
        out_type=[jax.ShapeDtypeStruct((NC, NS, CWB), jnp.int32),
                  jax.ShapeDtypeStruct((NC, NS, CWB), jnp.int32),
                  jax.ShapeDtypeStruct((NC, NS, 16), jnp.int32)],
        mesh=_mesh,
        scratch_types=[
            pltpu.VMEM((CWB,), jnp.int32),
            pltpu.VMEM((CWB,), jnp.int32),
            pltpu.VMEM((CWB + 544,), jnp.int32),
            pltpu.VMEM((CWB + 544,), jnp.int32),
            pltpu.VMEM((16,), jnp.int32),
        ],
        compiler_params=_cp,
    )(src, dst)


def _msg_body(y_hbm, srcc_hbm, dstc_hbm, ncnt_hbm, m_hbm,
              srcv, dstv, rows, nsm, acc, gsem, ssem):
    c = lax.axis_index("c")
    s = lax.axis_index("s")

    pltpu.sync_copy(ncnt_hbm.at[c, s], nsm)
    nch = nsm[pl.ds(0, 16)][0]
    pltpu.async_copy(srcc_hbm.at[c, s], srcv, gsem.at[0])
    pltpu.async_copy(dstc_hbm.at[c, s], dstv, gsem.at[1])

    @pl.loop(0, B)
    def _(r):
        @pl.loop(0, D, step=16)
        def _(k):
            rows[0, r, pl.ds(k, 16)] = jnp.zeros((16,), jnp.float32)

    pltpu.sync_copy(rows.at[0], acc.at[pl.ds(s * AZ, B)])
    pltpu.sync_copy(rows.at[0], acc.at[pl.ds(s * AZ + B, B)])
    pltpu.sync_copy(rows.at[0, pl.ds(0, AZ - 2 * B)],
                    acc.at[pl.ds(s * AZ + 2 * B, AZ - 2 * B)])
    pltpu.make_async_copy(srcc_hbm.at[c, s], srcv, gsem.at[0]).wait()
    pltpu.make_async_copy(dstc_hbm.at[c, s], dstv, gsem.at[1]).wait()
    plsc.subcore_barrier()

    for b in range(K):
        pltpu.async_copy(y_hbm.at[srcv.at[b]], rows.at[b], gsem.at[b])

    def ring(t, _):
        g = t * K
        for b in range(K):
            j = g + b
            pltpu.make_async_copy(y_hbm.at[srcv.at[j]], rows.at[b],
                                  gsem.at[b]).wait()
            pltpu.sync_copy(rows.at[b], acc.at[dstv.at[j]], add=True)
            nxt = jnp.minimum(g + K + b, nch - 1)
            pltpu.async_copy(y_hbm.at[srcv.at[nxt]], rows.at[b], gsem.at[b])
        return 0

    lax.fori_loop(0, nch // K, ring, 0)

    for b in range(K):
        pltpu.make_async_copy(y_hbm.at[srcv.at[0]], rows.at[b],
                              gsem.at[b]).wait()

    plsc.subcore_barrier()

    @pl.when(c == 0)
    def _():
        pltpu.sync_copy(acc.at[pl.ds(s * MR0, MR0)],
                        m_hbm.at[pl.ds(s * MR0, MR0)])

    @pl.when(c == 1)
    def _():
        pltpu.sync_copy(acc.at[pl.ds(s * MR1, MR1)],
                        m_hbm.at[pl.ds(H0 + s * MR1, MR1)])


@jax.jit
def _msg_call(y, srcc, dstc, ncnt):
    return pl.kernel(
        _msg_body,
        out_type=jax.ShapeDtypeStruct((NP, D), jnp.float32),
        mesh=_mesh,
        scratch_types=[
            pltpu.VMEM((CW, B), jnp.int32),
            pltpu.VMEM((CW, B), jnp.int32),
            pltpu.VMEM((K, B, D), jnp.float32),
            pltpu.VMEM((16,), jnp.int32),
            pltpu.VMEM_SHARED((AR, D), jnp.float32),
            pltpu.SemaphoreType.DMA((K,)),
            pltpu.SemaphoreType.DMA((K,)),
        ],
    )(y, srcc, dstc, ncnt)


R = 1024
EB = EP // (NP // R)


def _dinv(degp_ref):
    deg = degp_ref[0, :] + degp_ref[1, :] + 1.0
    return lax.rsqrt(jnp.maximum(deg, 1.0))


def _tc_first_body(x_ref, degp_ref, w_ref, y_ref):
    dinv = _dinv(degp_ref)
    xw = lax.dot_general(x_ref[...], w_ref[...], (((1,), (0,)), ((), ())),
                         preferred_element_type=jnp.float32)
    y_ref[...] = xw * dinv[:, None]


def _tc_mid_body(m_ref, yprev_ref, degp_ref, b_ref, w_ref, y_ref):
    dinv = _dinv(degp_ref)
    ssum = m_ref[...] + yprev_ref[...]
    h = jnp.maximum(ssum * dinv[:, None] + b_ref[...], 0.0)
    hw = lax.dot_general(h, w_ref[...], (((1,), (0,)), ((), ())),
                         preferred_element_type=jnp.float32)
    y_ref[...] = hw * dinv[:, None]


def _tc_last_body(m_ref, yprev_ref, degp_ref, b_ref, o_ref):
    dinv = _dinv(degp_ref)
    ssum = m_ref[...] + yprev_ref[...]
    o_ref[...] = ssum * dinv[:, None] + b_ref[...]


_row_spec = pl.BlockSpec((R, D), lambda i: (i, 0))
_deg_spec = pl.BlockSpec((NC, R), lambda i: (0, i))
_b_spec = pl.BlockSpec((1, D), lambda i: (0, 0))
_w_spec = pl.BlockSpec((D, D), lambda i: (0, 0))
_y_sds = jax.ShapeDtypeStruct((NP, D), jnp.float32)


@jax.jit
def _tc_first(xp, degp, W):
    return pl.pallas_call(
        _tc_first_body,
        grid=(NP // R,),
        in_specs=[_row_spec, _deg_spec, _w_spec],
        out_specs=_row_spec,
        out_shape=_y_sds,
    )(xp, degp, W)


@jax.jit
def _tc_mid(m, yprev, degp, b2d, W):
    return pl.pallas_call(
        _tc_mid_body,
        grid=(NP // R,),
        in_specs=[_row_spec, _row_spec, _deg_spec, _b_spec, _w_spec],
        out_specs=_row_spec,
        out_shape=_y_sds,
    )(m, yprev, degp, b2d, W)


@jax.jit
def _tc_last(m, yprev, degp, b2d):
    return pl.pallas_call(
        _tc_last_body,
        grid=(NP // R,),
        in_specs=[_row_spec, _row_spec, _deg_spec, _b_spec],
        out_specs=_row_spec,
        out_shape=_y_sds,
    )(m, yprev, degp, b2d)


def kernel(x, edge_index, W1, b1, W2, b2, W3, b3):
    xp = jnp.zeros((NP, D), jnp.float32).at[:N].set(x)
    pad = jnp.full((EP - E,), NP - 1, jnp.int32)
    src = jnp.concatenate([edge_index[0], pad])
    dst = jnp.concatenate([edge_index[1], pad])
    dst3 = dst.reshape(NW, C, B)

    degp = _deg_call(dst3)
    srcc, dstc, ncnt = _route_call(src, dst)
    srcc = srcc.reshape(NC, NS, CW, B)
    dstc = dstc.reshape(NC, NS, CW, B)

    y1 = _tc_first(xp, degp, W1)
    m1 = _msg_call(y1, srcc, dstc, ncnt)
    y2 = _tc_mid(m1, y1, degp, b1.reshape(1, D), W2)
    m2 = _msg_call(y2, srcc, dstc, ncnt)
    y3 = _tc_mid(m2, y2, degp, b2.reshape(1, D), W3)
    m3 = _msg_call(y3, srcc, dstc, ncnt)
    out = _tc_last(m3, y3, degp, b3.reshape(1, D))
    return out[:N]
